# speedup vs baseline: 1.8889x; 1.8889x over previous
import functools

import jax
import jax.numpy as jnp
from jax.experimental import pallas as pl
from jax.experimental.pallas import tpu as pltpu

_TOPK = 30
_NEG = -1e30


def _mlp_body(x, W1, b1, W2, b2, Wh, bh, Wt, bt, eh, et):
    h = jnp.maximum(jnp.dot(x[...], W1[...], preferred_element_type=jnp.float32) + b1[...], 0.0)
    h = jnp.maximum(jnp.dot(h, W2[...], preferred_element_type=jnp.float32) + b2[...], 0.0)
    eh[...] = jnp.dot(h, Wh[...], preferred_element_type=jnp.float32) + bh[...]
    et[...] = jnp.dot(h, Wt[...], preferred_element_type=jnp.float32) + bt[...]


def _logits_body(nreal, scale, eh, et, out):
    j = pl.program_id(1)
    cb = out.shape[1]
    l = jax.lax.dot_general(eh[...] * scale, et[...],
                            (((1,), (1,)), ((), ())),
                            preferred_element_type=jnp.float32)
    col = j * cb + jax.lax.broadcasted_iota(jnp.int32, l.shape, 1)
    out[...] = jnp.where(col < nreal, l, _NEG)


def _agg_body(eh, vals, nb, l1W, l1b, l2W, l2b, aW1, ab1, aW2, ab2,
              eh2_out, att_out):
    e = eh[...]
    v = vals[...]
    nbv = nb[...]
    p = jax.nn.softmax(v, axis=-1)
    s = jnp.sum(nbv, axis=-1)
    arg = (2.0 - p)[:, :, None] * e[:, None, :] + p[:, :, None] * nbv
    g = jnp.sum(jnp.tanh(arg), axis=-1)
    ka = s * g
    q = jax.nn.softmax(ka, axis=-1)
    eNh = jnp.sum(q[:, :, None] * nbv, axis=1)
    sum_emb = jnp.maximum(
        jnp.dot(e + eNh, l1W[...], preferred_element_type=jnp.float32) + l1b[...], 0.0)
    bi_emb = jnp.maximum(
        jnp.dot(e * eNh, l2W[...], preferred_element_type=jnp.float32) + l2b[...], 0.0)
    eh2 = sum_emb + bi_emb
    pre = jnp.dot(eh2, aW1[...], preferred_element_type=jnp.float32) + ab1[...]
    act = jnp.where(pre > 0, pre, 0.01 * pre)
    a = jnp.dot(act, aW2[...], preferred_element_type=jnp.float32) + ab2[...]
    eh2_out[...] = eh2
    att_out[...] = a


def _readout_body(nreal, att, eh2, out):
    row = jax.lax.broadcasted_iota(jnp.int32, att.shape, 0)
    a = jnp.where(row < nreal, att[...], _NEG)
    m = jnp.max(a)
    ex = jnp.exp(a - m)
    alpha = ex / jnp.sum(ex)
    out[...] = jnp.sum(alpha * eh2[...], axis=0, keepdims=True)


def kernel(x_path, fc_W1, fc_b1, fc_W2, fc_b2, Wh, bh, Wt, bt,
           l1_W, l1_b, l2_W, l2_b, att_W1, att_b1, att_W2, att_b2):
    n, din = x_path.shape
    d = fc_W1.shape[1]
    dh = att_W1.shape[1]
    scale = d ** (-0.5)

    CB = 512
    npad = -(-n // CB) * CB
    RB_MLP = 512
    RB_LG = 256
    RB_AG = 256

    x_p = jnp.pad(x_path, ((0, npad - n), (0, 0)))
    b1 = fc_b1.reshape(1, d)
    b2 = fc_b2.reshape(1, d)
    bh_ = bh.reshape(1, d)
    bt_ = bt.reshape(1, d)

    eh, et = pl.pallas_call(
        _mlp_body,
        grid=(npad // RB_MLP,),
        in_specs=[
            pl.BlockSpec((RB_MLP, din), lambda i: (i, 0)),
            pl.BlockSpec((din, d), lambda i: (0, 0)),
            pl.BlockSpec((1, d), lambda i: (0, 0)),
            pl.BlockSpec((d, d), lambda i: (0, 0)),
            pl.BlockSpec((1, d), lambda i: (0, 0)),
            pl.BlockSpec((d, d), lambda i: (0, 0)),
            pl.BlockSpec((1, d), lambda i: (0, 0)),
            pl.BlockSpec((d, d), lambda i: (0, 0)),
            pl.BlockSpec((1, d), lambda i: (0, 0)),
        ],
        out_specs=[
            pl.BlockSpec((RB_MLP, d), lambda i: (i, 0)),
            pl.BlockSpec((RB_MLP, d), lambda i: (i, 0)),
        ],
        out_shape=[
            jax.ShapeDtypeStruct((npad, d), jnp.float32),
            jax.ShapeDtypeStruct((npad, d), jnp.float32),
        ],
    )(x_p, fc_W1, b1, fc_W2, b2, Wh, bh_, Wt, bt_)

    logits = pl.pallas_call(
        functools.partial(_logits_body, n, scale),
        grid=(npad // RB_LG, npad // CB),
        in_specs=[
            pl.BlockSpec((RB_LG, d), lambda i, j: (i, 0)),
            pl.BlockSpec((CB, d), lambda i, j: (j, 0)),
        ],
        out_specs=pl.BlockSpec((RB_LG, CB), lambda i, j: (i, j)),
        out_shape=jax.ShapeDtypeStruct((npad, npad), jnp.float32),
    )(eh, et)

    vals, idx = jax.lax.top_k(logits, _TOPK)
    nb = jnp.take(et, idx, axis=0)

    l1b = l1_b.reshape(1, d)
    l2b = l2_b.reshape(1, d)
    ab1 = att_b1.reshape(1, dh)
    ab2 = att_b2.reshape(1, 1)

    eh2, att = pl.pallas_call(
        _agg_body,
        grid=(npad // RB_AG,),
        in_specs=[
            pl.BlockSpec((RB_AG, d), lambda i: (i, 0)),
            pl.BlockSpec((RB_AG, _TOPK), lambda i: (i, 0)),
            pl.BlockSpec((RB_AG, _TOPK, d), lambda i: (i, 0, 0)),
            pl.BlockSpec((d, d), lambda i: (0, 0)),
            pl.BlockSpec((1, d), lambda i: (0, 0)),
            pl.BlockSpec((d, d), lambda i: (0, 0)),
            pl.BlockSpec((1, d), lambda i: (0, 0)),
            pl.BlockSpec((d, dh), lambda i: (0, 0)),
            pl.BlockSpec((1, dh), lambda i: (0, 0)),
            pl.BlockSpec((dh, 1), lambda i: (0, 0)),
            pl.BlockSpec((1, 1), lambda i: (0, 0)),
        ],
        out_specs=[
            pl.BlockSpec((RB_AG, d), lambda i: (i, 0)),
            pl.BlockSpec((RB_AG, 1), lambda i: (i, 0)),
        ],
        out_shape=[
            jax.ShapeDtypeStruct((npad, d), jnp.float32),
            jax.ShapeDtypeStruct((npad, 1), jnp.float32),
        ],
    )(eh, vals, nb, l1_W, l1b, l2_W, l2b, att_W1, ab1, att_W2, ab2)

    e_g = pl.pallas_call(
        functools.partial(_readout_body, n),
        in_specs=[
            pl.BlockSpec((npad, 1), lambda: (0, 0)),
            pl.BlockSpec((npad, d), lambda: (0, 0)),
        ],
        out_specs=pl.BlockSpec((1, d), lambda: (0, 0)),
        out_shape=jax.ShapeDtypeStruct((1, d), jnp.float32),
    )(att, eh2)

    return e_g


# speedup vs baseline: 4.4894x; 2.3767x over previous
import functools

import jax
import jax.numpy as jnp
from jax import lax
from jax.experimental import pallas as pl
from jax.experimental.pallas import tpu as pltpu
from jax.experimental.pallas import tpu_sc as plsc

_TOPK = 30
_KPAD = 32
_NEG = -1e30


def _mlp_body(x, W1, b1, W2, b2, Wh, bh, Wt, bt, eh, et):
    h = jnp.maximum(jnp.dot(x[...], W1[...], preferred_element_type=jnp.float32) + b1[...], 0.0)
    h = jnp.maximum(jnp.dot(h, W2[...], preferred_element_type=jnp.float32) + b2[...], 0.0)
    eh[...] = jnp.dot(h, Wh[...], preferred_element_type=jnp.float32) + bh[...]
    et[...] = jnp.dot(h, Wt[...], preferred_element_type=jnp.float32) + bt[...]


def _logits_body(nreal, scale, eh, et, out):
    j = pl.program_id(1)
    cb = out.shape[1]
    l = jax.lax.dot_general(eh[...] * scale, et[...],
                            (((1,), (1,)), ((), ())),
                            preferred_element_type=jnp.float32)
    col = j * cb + jax.lax.broadcasted_iota(jnp.int32, l.shape, 1)
    out[...] = jnp.where(col < nreal, l, _NEG)


def _agg_body(eh, vals, nb, l1W, l1b, l2W, l2b, aW1, ab1, aW2, ab2,
              eh2_out, att_out):
    e = eh[...]
    v = vals[...]
    nbv = nb[...]
    kio = jax.lax.broadcasted_iota(jnp.int32, v.shape, 1)
    kmask = kio < _TOPK
    v = jnp.where(kmask, v, _NEG)
    p = jax.nn.softmax(v, axis=-1)
    s = jnp.sum(nbv, axis=-1)
    arg = (2.0 - p)[:, :, None] * e[:, None, :] + p[:, :, None] * nbv
    g = jnp.sum(jnp.tanh(arg), axis=-1)
    ka = jnp.where(kmask, s * g, _NEG)
    q = jax.nn.softmax(ka, axis=-1)
    eNh = jnp.sum(q[:, :, None] * nbv, axis=1)
    sum_emb = jnp.maximum(
        jnp.dot(e + eNh, l1W[...], preferred_element_type=jnp.float32) + l1b[...], 0.0)
    bi_emb = jnp.maximum(
        jnp.dot(e * eNh, l2W[...], preferred_element_type=jnp.float32) + l2b[...], 0.0)
    eh2 = sum_emb + bi_emb
    pre = jnp.dot(eh2, aW1[...], preferred_element_type=jnp.float32) + ab1[...]
    act = jnp.where(pre > 0, pre, 0.01 * pre)
    a = jnp.dot(act, aW2[...], preferred_element_type=jnp.float32) + ab2[...]
    eh2_out[...] = eh2
    att_out[...] = a


def _merge2(av, ai, bv, bi):
    rbv = lax.rev(bv, (0,))
    rbi = lax.rev(bi, (0,))
    m = av >= rbv
    pv = jnp.where(m, av, rbv)
    pi = jnp.where(m, ai, rbi)
    qv = jnp.where(m, rbv, av)
    qi = jnp.where(m, rbi, ai)
    pv, pi = plsc.sort_key_val(pv, pi, descending=True)
    qv, qi = plsc.sort_key_val(qv, qi, descending=True)
    return pv, pi, qv, qi


def _make_sc_topk(npad):
    info = plsc.get_sparse_core_info()
    nw = info.num_cores * info.num_subcores
    rows_per_w = npad // nw
    ngroups = npad // 128
    mesh = plsc.VectorSubcoreMesh(core_axis_name="c", subcore_axis_name="s")

    @functools.partial(
        pl.kernel, mesh=mesh,
        compiler_params=pltpu.CompilerParams(needs_layout_passes=False),
        out_type=[jax.ShapeDtypeStruct((npad, _KPAD), jnp.float32),
                  jax.ShapeDtypeStruct((npad, _KPAD), jnp.int32)],
        scratch_types=[
            pltpu.VMEM((8, npad), jnp.float32),
            pltpu.VMEM((2, 16), jnp.float32),
            pltpu.VMEM((2, 16), jnp.int32),
            pltpu.VMEM((8, _KPAD), jnp.float32),
            pltpu.VMEM((8, _KPAD), jnp.int32),
            pltpu.SMEM((1,), jnp.float32),
        ],
    )
    def topk_k(logits_hbm, vals_hbm, idx_hbm, rbuf, bv, bi, obv, obi, thr):
        wid = lax.axis_index("s") * info.num_cores + lax.axis_index("c")
        row0 = wid * rows_per_w
        io16 = lax.iota(jnp.int32, 16)

        def row8_body(r8, carry):
            gr0 = row0 + r8 * 8
            pltpu.sync_copy(logits_hbm.at[pl.ds(gr0, 8)], rbuf)

            def row_body(j, c1):
                v0, i0 = plsc.sort_key_val(rbuf[j, pl.ds(0, 16)], io16,
                                           descending=True)
                v1, i1 = plsc.sort_key_val(rbuf[j, pl.ds(16, 16)], io16 + 16,
                                           descending=True)
                hv, hi, lv, li = _merge2(v0, i0, v1, i1)
                for k in range(2, 8):
                    cs, cis = plsc.sort_key_val(rbuf[j, pl.ds(k * 16, 16)],
                                                io16 + k * 16, descending=True)
                    hv, hi, mv, mi = _merge2(cs, cis, hv, hi)
                    lv, li, _, _ = _merge2(mv, mi, lv, li)
                bv[0] = hv
                bi[0] = hi
                bv[1] = lv
                bi[1] = li
                thr[0] = jnp.min(lv)

                def group_body(g, c2):
                    base = g * 128
                    m = rbuf[j, pl.ds(base, 16)]
                    for k in range(1, 8):
                        m = jnp.maximum(m, rbuf[j, pl.ds(base + k * 16, 16)])

                    @pl.when(jnp.max(m) > thr[0])
                    def _slow():
                        for k in range(8):
                            cv = rbuf[j, pl.ds(base + k * 16, 16)]

                            @pl.when(jnp.max(cv) > thr[0])
                            def _merge_chunk():
                                cs, cis = plsc.sort_key_val(
                                    cv, io16 + (base + k * 16),
                                    descending=True)
                                hv2, hi2, mv2, mi2 = _merge2(
                                    cs, cis, bv[0], bi[0])
                                lv2, li2, _, _ = _merge2(
                                    mv2, mi2, bv[1], bi[1])
                                bv[0] = hv2
                                bi[0] = hi2
                                bv[1] = lv2
                                bi[1] = li2
                                thr[0] = jnp.min(lv2)
                    return c2

                lax.fori_loop(1, ngroups, group_body, 0)

                obv[j, pl.ds(0, 16)] = bv[0]
                obv[j, pl.ds(16, 16)] = bv[1]
                obi[j, pl.ds(0, 16)] = bi[0]
                obi[j, pl.ds(16, 16)] = bi[1]
                return c1

            lax.fori_loop(0, 8, row_body, 0)
            pltpu.sync_copy(obv, vals_hbm.at[pl.ds(gr0, 8)])
            pltpu.sync_copy(obi, idx_hbm.at[pl.ds(gr0, 8)])
            return carry

        lax.fori_loop(0, rows_per_w // 8, row8_body, 0)

    return topk_k


def _make_sc_gather(npad, d):
    info = plsc.get_sparse_core_info()
    nw = info.num_cores * info.num_subcores
    total = npad * _KPAD
    per_w = total // nw
    chunk = 128
    nchunks = per_w // chunk

    mesh = plsc.VectorSubcoreMesh(core_axis_name="c", subcore_axis_name="s")

    @functools.partial(
        pl.kernel, mesh=mesh,
        out_type=jax.ShapeDtypeStruct((total, d), jnp.float32),
        scratch_types=[
            pltpu.VMEM((chunk,), jnp.int32),
            pltpu.VMEM((chunk, d), jnp.float32),
            pltpu.SemaphoreType.DMA,
        ],
    )
    def gather_k(et_hbm, idx_hbm, out_hbm, idx_v, rows_v, sem):
        wid = lax.axis_index("s") * info.num_cores + lax.axis_index("c")
        base = wid * per_w

        def body(j, carry):
            off = base + j * chunk
            pltpu.sync_copy(idx_hbm.at[pl.ds(off, chunk)], idx_v)
            pltpu.async_copy(et_hbm.at[idx_v], rows_v, sem).wait()
            pltpu.sync_copy(rows_v, out_hbm.at[pl.ds(off, chunk)])
            return carry

        lax.fori_loop(0, nchunks, body, 0)

    return gather_k


def _readout_body(nreal, att, eh2, out):
    row = jax.lax.broadcasted_iota(jnp.int32, att.shape, 0)
    a = jnp.where(row < nreal, att[...], _NEG)
    m = jnp.max(a)
    ex = jnp.exp(a - m)
    alpha = ex / jnp.sum(ex)
    out[...] = jnp.sum(alpha * eh2[...], axis=0, keepdims=True)


def kernel(x_path, fc_W1, fc_b1, fc_W2, fc_b2, Wh, bh, Wt, bt,
           l1_W, l1_b, l2_W, l2_b, att_W1, att_b1, att_W2, att_b2):
    n, din = x_path.shape
    d = fc_W1.shape[1]
    dh = att_W1.shape[1]
    scale = d ** (-0.5)

    CB = 512
    npad = -(-n // CB) * CB
    RB_MLP = 512
    RB_LG = 256
    RB_AG = 256

    x_p = jnp.pad(x_path, ((0, npad - n), (0, 0)))
    b1 = fc_b1.reshape(1, d)
    b2 = fc_b2.reshape(1, d)
    bh_ = bh.reshape(1, d)
    bt_ = bt.reshape(1, d)

    eh, et = pl.pallas_call(
        _mlp_body,
        grid=(npad // RB_MLP,),
        in_specs=[
            pl.BlockSpec((RB_MLP, din), lambda i: (i, 0)),
            pl.BlockSpec((din, d), lambda i: (0, 0)),
            pl.BlockSpec((1, d), lambda i: (0, 0)),
            pl.BlockSpec((d, d), lambda i: (0, 0)),
            pl.BlockSpec((1, d), lambda i: (0, 0)),
            pl.BlockSpec((d, d), lambda i: (0, 0)),
            pl.BlockSpec((1, d), lambda i: (0, 0)),
            pl.BlockSpec((d, d), lambda i: (0, 0)),
            pl.BlockSpec((1, d), lambda i: (0, 0)),
        ],
        out_specs=[
            pl.BlockSpec((RB_MLP, d), lambda i: (i, 0)),
            pl.BlockSpec((RB_MLP, d), lambda i: (i, 0)),
        ],
        out_shape=[
            jax.ShapeDtypeStruct((npad, d), jnp.float32),
            jax.ShapeDtypeStruct((npad, d), jnp.float32),
        ],
    )(x_p, fc_W1, b1, fc_W2, b2, Wh, bh_, Wt, bt_)

    logits = pl.pallas_call(
        functools.partial(_logits_body, n, scale),
        grid=(npad // RB_LG, npad // CB),
        in_specs=[
            pl.BlockSpec((RB_LG, d), lambda i, j: (i, 0)),
            pl.BlockSpec((CB, d), lambda i, j: (j, 0)),
        ],
        out_specs=pl.BlockSpec((RB_LG, CB), lambda i, j: (i, j)),
        out_shape=jax.ShapeDtypeStruct((npad, npad), jnp.float32),
    )(eh, et)

    vals, idx = _make_sc_topk(npad)(logits)
    idx_flat = idx.reshape(npad * _KPAD)
    nb_flat = _make_sc_gather(npad, d)(et, idx_flat)
    nb = nb_flat.reshape(npad, _KPAD, d)

    l1b = l1_b.reshape(1, d)
    l2b = l2_b.reshape(1, d)
    ab1 = att_b1.reshape(1, dh)
    ab2 = att_b2.reshape(1, 1)

    eh2, att = pl.pallas_call(
        _agg_body,
        grid=(npad // RB_AG,),
        in_specs=[
            pl.BlockSpec((RB_AG, d), lambda i: (i, 0)),
            pl.BlockSpec((RB_AG, _KPAD), lambda i: (i, 0)),
            pl.BlockSpec((RB_AG, _KPAD, d), lambda i: (i, 0, 0)),
            pl.BlockSpec((d, d), lambda i: (0, 0)),
            pl.BlockSpec((1, d), lambda i: (0, 0)),
            pl.BlockSpec((d, d), lambda i: (0, 0)),
            pl.BlockSpec((1, d), lambda i: (0, 0)),
            pl.BlockSpec((d, dh), lambda i: (0, 0)),
            pl.BlockSpec((1, dh), lambda i: (0, 0)),
            pl.BlockSpec((dh, 1), lambda i: (0, 0)),
            pl.BlockSpec((1, 1), lambda i: (0, 0)),
        ],
        out_specs=[
            pl.BlockSpec((RB_AG, d), lambda i: (i, 0)),
            pl.BlockSpec((RB_AG, 1), lambda i: (i, 0)),
        ],
        out_shape=[
            jax.ShapeDtypeStruct((npad, d), jnp.float32),
            jax.ShapeDtypeStruct((npad, 1), jnp.float32),
        ],
    )(eh, vals, nb, l1_W, l1b, l2_W, l2b, att_W1, ab1, att_W2, ab2)

    e_g = pl.pallas_call(
        functools.partial(_readout_body, n),
        in_specs=[
            pl.BlockSpec((npad, 1), lambda: (0, 0)),
            pl.BlockSpec((npad, d), lambda: (0, 0)),
        ],
        out_specs=pl.BlockSpec((1, d), lambda: (0, 0)),
        out_shape=jax.ShapeDtypeStruct((1, d), jnp.float32),
    )(att, eh2)

    return e_g
